# baseline (speedup 1.0000x reference)
import functools

import jax
import jax.numpy as jnp
from jax import lax
from jax.experimental import pallas as pl
from jax.experimental.pallas import tpu as pltpu
from jax.experimental.pallas import tpu_sc as plsc

_NC = 2
_NS = 16
_L = 16
_NW = _NC * _NS


def _ef_body(x_ref, std_ref, mean_ref, out_ref):
    vm = x_ref[:, 0:1] * std_ref[0, 0] + mean_ref[0, 0]
    va = (x_ref[:, 1:2] * std_ref[0, 1] + mean_ref[0, 1]) * (jnp.pi / 180.0)
    out_ref[:, 0:1] = vm * jnp.cos(va)
    out_ref[:, 1:2] = vm * jnp.sin(va)


def _ef_table(x, xystd, xymean):
    n = x.shape[0]
    bn = 10000
    return pl.pallas_call(
        _ef_body,
        grid=(n // bn,),
        in_specs=[
            pl.BlockSpec((bn, 6), lambda i: (i, 0)),
            pl.BlockSpec((1, 6), lambda i: (0, 0)),
            pl.BlockSpec((1, 6), lambda i: (0, 0)),
        ],
        out_specs=pl.BlockSpec((bn, 2), lambda i: (i, 0)),
        out_shape=jax.ShapeDtypeStruct((n, 2), jnp.float32),
    )(x, xystd, xymean)


def _sc_edge_pass(ep_tab, eif, ea, prm_f, prm_i):
    n = ep_tab.shape[0]
    e_total = eif.shape[0] // 2
    epw = e_total // _NW
    c_chunk = 128
    n_full = (epw - 80) // c_chunk
    tail = epw - n_full * c_chunk
    zsl = ((n + _NS * _L - 1) // (_NS * _L)) * _L
    zlast = n - (_NS - 1) * zsl

    mesh = plsc.VectorSubcoreMesh(core_axis_name="c", subcore_axis_name="s")

    @functools.partial(
        pl.kernel,
        out_type=(
            jax.ShapeDtypeStruct((_NC * n,), jnp.float32),
            jax.ShapeDtypeStruct((_NC * n,), jnp.float32),
            jax.ShapeDtypeStruct((_NC * n,), jnp.float32),
            jax.ShapeDtypeStruct((_NC * n,), jnp.float32),
            jax.ShapeDtypeStruct((_NW * _L,), jnp.int32),
        ),
        mesh=mesh,
        compiler_params=pltpu.CompilerParams(needs_layout_passes=False),
        scratch_types=[
            pltpu.VMEM_SHARED((n,), jnp.float32),
            pltpu.VMEM_SHARED((n,), jnp.float32),
            pltpu.VMEM_SHARED((n,), jnp.float32),
            pltpu.VMEM_SHARED((n,), jnp.float32),
            pltpu.VMEM_SHARED((n,), jnp.float32),
            pltpu.VMEM((zsl,), jnp.float32),
            pltpu.VMEM((zsl,), jnp.float32),
            pltpu.VMEM((2, c_chunk), jnp.int32),
            pltpu.VMEM((2, c_chunk), jnp.int32),
            pltpu.VMEM((2, c_chunk, 2), jnp.float32),
            pltpu.VMEM((80,), jnp.int32),
            pltpu.VMEM((80,), jnp.int32),
            pltpu.VMEM((c_chunk,), jnp.float32),
            pltpu.VMEM((c_chunk,), jnp.float32),
            pltpu.VMEM((c_chunk,), jnp.float32),
            pltpu.VMEM((c_chunk,), jnp.float32),
            pltpu.VMEM((c_chunk,), jnp.float32),
            pltpu.VMEM((c_chunk,), jnp.float32),
            pltpu.VMEM((_L,), jnp.int32),
            pltpu.VMEM((8,), jnp.float32),
            pltpu.VMEM((8,), jnp.int32),
            pltpu.SemaphoreType.DMA,
            pltpu.SemaphoreType.DMA,
            pltpu.SemaphoreType.DMA,
        ],
    )
    def sc_kernel(ep_hbm, eif_hbm, ea_hbm,
                  prmf_hbm, prmi_hbm,
                  aggpo_hbm, aggqo_hbm, aggpd_hbm, aggqd_hbm, flags_hbm,
                  ep_sh, po_sh, qo_sh, pd_sh, qd_sh, zbuf, bbuf,
                  ii2, jj2, ea2, ii_t, jj_t, gvi, gvj,
                  p1_v, q1_v, p2_v, q2_v, flags_v, prmf_v, prmi_v,
                  semg, seml0, seml1):
        cc = lax.axis_index("c")
        ss = lax.axis_index("s")
        wid = ss * _NC + cc
        z16i = jnp.zeros((_L,), jnp.int32)
        one16i = z16i + 1
        zero16f = jnp.zeros((_L,), jnp.float32)
        iota16 = lax.iota(jnp.int32, _L)
        seml = (seml0, seml1)

        @pl.loop(0, zsl // _L)
        def _(zi):
            zbuf[pl.ds(zi * _L, _L)] = zero16f

        off = ss * zsl

        @pl.when(ss < _NS - 1)
        def _():
            pltpu.sync_copy(ep_hbm.at[pl.ds(off, zsl)], bbuf)
            pltpu.sync_copy(bbuf, ep_sh.at[pl.ds(off, zsl)])
            pltpu.sync_copy(zbuf, po_sh.at[pl.ds(off, zsl)])
            pltpu.sync_copy(zbuf, qo_sh.at[pl.ds(off, zsl)])
            pltpu.sync_copy(zbuf, pd_sh.at[pl.ds(off, zsl)])
            pltpu.sync_copy(zbuf, qd_sh.at[pl.ds(off, zsl)])

        @pl.when(ss == _NS - 1)
        def _():
            pltpu.sync_copy(ep_hbm.at[pl.ds(off, zlast)], bbuf.at[pl.ds(0, zlast)])
            pltpu.sync_copy(bbuf.at[pl.ds(0, zlast)], ep_sh.at[pl.ds(off, zlast)])
            pltpu.sync_copy(zbuf.at[pl.ds(0, zlast)], po_sh.at[pl.ds(off, zlast)])
            pltpu.sync_copy(zbuf.at[pl.ds(0, zlast)], qo_sh.at[pl.ds(off, zlast)])
            pltpu.sync_copy(zbuf.at[pl.ds(0, zlast)], pd_sh.at[pl.ds(off, zlast)])
            pltpu.sync_copy(zbuf.at[pl.ds(0, zlast)], qd_sh.at[pl.ds(off, zlast)])

        pltpu.sync_copy(prmf_hbm, prmf_v)
        pltpu.sync_copy(prmi_hbm, prmi_v)
        estd0 = plsc.load_gather(prmf_v, [one16i])
        estd1 = plsc.load_gather(prmf_v, [one16i + 1])
        emn0 = plsc.load_gather(prmf_v, [one16i + 2])
        emn1 = plsc.load_gather(prmf_v, [one16i + 3])
        a0 = plsc.load_gather(prmi_v, [one16i])
        b0 = plsc.load_gather(prmi_v, [one16i + 1])
        flags_v[...] = z16i

        plsc.subcore_barrier()

        base_w = wid * epw

        def compute_scatter(iref, jref, earef, cs):
            g1 = pltpu.async_copy(ep_sh.at[iref], gvi.at[pl.ds(0, cs)], semg)
            g2 = pltpu.async_copy(ep_sh.at[jref], gvj.at[pl.ds(0, cs)], semg)
            g1.wait()
            g2.wait()
            for kk in range(cs // _L):
                sl = pl.ds(kk * _L, _L)
                lidx = kk * _L + iota16
                src16 = iref[sl]
                dst16 = jref[sl]
                e_i, f_i = plsc.unpack(
                    plsc.bitcast(gvi[sl], jnp.bfloat16),
                    format=plsc.PackFormat.INTERLEAVED)
                e_j, f_j = plsc.unpack(
                    plsc.bitcast(gvj[sl], jnp.bfloat16),
                    format=plsc.PackFormat.INTERLEAVED)
                r = plsc.load_gather(earef, [lidx, z16i]) * estd0 + emn0
                xx = plsc.load_gather(earef, [lidx, one16i]) * estd1 + emn1
                inv = 1.0 / (r * r + xx * xx)
                g = r * inv
                b = -(xx * inv)
                eef = e_i * e_j + f_i * f_j
                tb = f_i * e_j - e_i * f_j
                ta = eef - e_i * e_i - f_i * f_i
                ta2 = eef - e_j * e_j - f_j * f_j
                gtb = g * tb
                btb = b * tb
                p1_v[sl] = g * ta + btb
                q1_v[sl] = gtb - b * ta
                p2_v[sl] = g * ta2 - btb
                q2_v[sl] = -gtb - b * ta2
                hit = (src16 == b0) & (dst16 == a0)
                flags_v[...] = flags_v[...] | jnp.where(hit, 1, 0)
            s1 = pltpu.async_copy(p1_v.at[pl.ds(0, cs)], po_sh.at[iref],
                                  semg, add=True)
            s2 = pltpu.async_copy(q1_v.at[pl.ds(0, cs)], qo_sh.at[iref],
                                  semg, add=True)
            s3 = pltpu.async_copy(p2_v.at[pl.ds(0, cs)], pd_sh.at[jref],
                                  semg, add=True)
            s4 = pltpu.async_copy(q2_v.at[pl.ds(0, cs)], qd_sh.at[jref],
                                  semg, add=True)
            s1.wait()
            s2.wait()
            s3.wait()
            s4.wait()

        pltpu.sync_copy(eif_hbm.at[pl.ds(base_w, tail)], ii_t)
        pltpu.sync_copy(eif_hbm.at[pl.ds(e_total + base_w, tail)], jj_t)
        pltpu.sync_copy(ea_hbm.at[pl.ds(base_w, tail)],
                        ea2.at[0].at[pl.ds(0, tail)])
        compute_scatter(ii_t, jj_t, ea2.at[0], tail)

        base0 = base_w + tail

        def issue_linear(k, bb):
            base = base0 + k * c_chunk
            pltpu.async_copy(eif_hbm.at[pl.ds(base, c_chunk)], ii2.at[bb],
                             seml[bb])
            pltpu.async_copy(eif_hbm.at[pl.ds(e_total + base, c_chunk)],
                             jj2.at[bb], seml[bb])
            pltpu.async_copy(ea_hbm.at[pl.ds(base, c_chunk)], ea2.at[bb],
                             seml[bb])

        def drain_linear(bb):
            pltpu.make_async_copy(eif_hbm.at[pl.ds(0, c_chunk)], ii2.at[bb],
                                  seml[bb]).wait()
            pltpu.make_async_copy(eif_hbm.at[pl.ds(0, c_chunk)], jj2.at[bb],
                                  seml[bb]).wait()
            pltpu.make_async_copy(ea_hbm.at[pl.ds(0, c_chunk)],
                                  ea2.at[bb], seml[bb]).wait()

        issue_linear(0, 0)

        @pl.loop(0, n_full // 2)
        def _(it):
            for bb in range(2):
                k = it * 2 + bb
                drain_linear(bb)

                @pl.when(k + 1 < n_full)
                def _():
                    issue_linear(k + 1, 1 - bb)

                compute_scatter(ii2.at[bb], jj2.at[bb], ea2.at[bb], c_chunk)

        pltpu.sync_copy(flags_v, flags_hbm.at[pl.ds(wid * _L, _L)])
        plsc.subcore_barrier()

        hoff = cc * n + off

        @pl.when(ss < _NS - 1)
        def _():
            pltpu.sync_copy(po_sh.at[pl.ds(off, zsl)], bbuf)
            pltpu.sync_copy(bbuf, aggpo_hbm.at[pl.ds(hoff, zsl)])
            pltpu.sync_copy(qo_sh.at[pl.ds(off, zsl)], bbuf)
            pltpu.sync_copy(bbuf, aggqo_hbm.at[pl.ds(hoff, zsl)])
            pltpu.sync_copy(pd_sh.at[pl.ds(off, zsl)], bbuf)
            pltpu.sync_copy(bbuf, aggpd_hbm.at[pl.ds(hoff, zsl)])
            pltpu.sync_copy(qd_sh.at[pl.ds(off, zsl)], bbuf)
            pltpu.sync_copy(bbuf, aggqd_hbm.at[pl.ds(hoff, zsl)])

        @pl.when(ss == _NS - 1)
        def _():
            pltpu.sync_copy(po_sh.at[pl.ds(off, zlast)], bbuf.at[pl.ds(0, zlast)])
            pltpu.sync_copy(bbuf.at[pl.ds(0, zlast)], aggpo_hbm.at[pl.ds(hoff, zlast)])
            pltpu.sync_copy(qo_sh.at[pl.ds(off, zlast)], bbuf.at[pl.ds(0, zlast)])
            pltpu.sync_copy(bbuf.at[pl.ds(0, zlast)], aggqo_hbm.at[pl.ds(hoff, zlast)])
            pltpu.sync_copy(pd_sh.at[pl.ds(off, zlast)], bbuf.at[pl.ds(0, zlast)])
            pltpu.sync_copy(bbuf.at[pl.ds(0, zlast)], aggpd_hbm.at[pl.ds(hoff, zlast)])
            pltpu.sync_copy(qd_sh.at[pl.ds(off, zlast)], bbuf.at[pl.ds(0, zlast)])
            pltpu.sync_copy(bbuf.at[pl.ds(0, zlast)], aggqd_hbm.at[pl.ds(hoff, zlast)])

    return sc_kernel(ep_tab, eif, ea, prm_f, prm_i)


def _loss_body(w_ref, flags_ref, std_ref, mean_ref, out_ref, *, n_nodes):
    pid = pl.program_id(0)

    @pl.when(pid == 0)
    def _():
        out_ref[...] = jnp.zeros((1, 1), jnp.float32)

    wb = w_ref[...]
    xs = wb[:, 0:6]
    ys = wb[:, 6:12]
    mse_part = jnp.sum((xs - ys) ** 2, keepdims=True)
    wdup = jnp.where(jnp.any(flags_ref[...] != 0), 0.0, 1.0)
    agg_p = wb[:, 12:13] + wb[:, 13:14] + wdup * (wb[:, 16:17] + wb[:, 17:18])
    agg_q = wb[:, 14:15] + wb[:, 15:16] + wdup * (wb[:, 18:19] + wb[:, 19:20])
    p_inj = wb[:, 2:3] * std_ref[0, 2] + mean_ref[0, 2]
    q_inj = wb[:, 3:4] * std_ref[0, 3] + mean_ref[0, 3]
    dp = p_inj - agg_p
    dq = q_inj - agg_q
    imb_part = jnp.sum(dp * dp + dq * dq, keepdims=True)
    out_ref[...] += (0.5 / (6.0 * n_nodes)) * mse_part + \
        (0.5 * 0.02 / n_nodes) * imb_part


def _loss_kernel(w, flags, xystd, xymean):
    n = w.shape[0]
    bn = 10000
    return pl.pallas_call(
        functools.partial(_loss_body, n_nodes=n),
        grid=(n // bn,),
        in_specs=[
            pl.BlockSpec((bn, 20), lambda i: (i, 0)),
            pl.BlockSpec((_NW * _L // 128, 128), lambda i: (0, 0)),
            pl.BlockSpec((1, 6), lambda i: (0, 0)),
            pl.BlockSpec((1, 6), lambda i: (0, 0)),
        ],
        out_specs=pl.BlockSpec((1, 1), lambda i: (0, 0)),
        out_shape=jax.ShapeDtypeStruct((1, 1), jnp.float32),
    )(w, flags, xystd, xymean)


def kernel(x, edge_index, edge_attr, y, xymean, xystd, edgemean, edgestd):
    ef = _ef_table(x, xystd, xymean)
    zf1 = jnp.zeros((1,), jnp.float32)
    prm_f = jnp.concatenate([zf1, edgestd.reshape(2), edgemean.reshape(2),
                             jnp.zeros((3,), jnp.float32)])
    zi1 = jnp.zeros((1,), jnp.int32)
    prm_i = jnp.concatenate([zi1, edge_index[0, 0:1], edge_index[1, 0:1],
                             jnp.zeros((5,), jnp.int32)])
    eb = lax.bitcast_convert_type(
        ef[:, 0].astype(jnp.bfloat16), jnp.uint16).astype(jnp.uint32)
    fb = lax.bitcast_convert_type(
        ef[:, 1].astype(jnp.bfloat16), jnp.uint16).astype(jnp.uint32)
    ep = lax.bitcast_convert_type(eb | (fb << jnp.uint32(16)), jnp.float32)
    aggpo, aggqo, aggpd, aggqd, flags = _sc_edge_pass(
        ep, edge_index.reshape(-1), edge_attr, prm_f, prm_i)
    w = jnp.concatenate(
        [x, y,
         aggpo.reshape(_NC, -1).T, aggqo.reshape(_NC, -1).T,
         aggpd.reshape(_NC, -1).T, aggqd.reshape(_NC, -1).T], axis=1)
    loss = _loss_kernel(w, flags.reshape(_NW * _L // 128, 128),
                        xystd, xymean)
    return loss[0, 0]


# speedup vs baseline: 1.6940x; 1.6940x over previous
import functools

import jax
import jax.numpy as jnp
from jax import lax
from jax.experimental import pallas as pl
from jax.experimental.pallas import tpu as pltpu
from jax.experimental.pallas import tpu_sc as plsc

_NC = 2
_NS = 16
_L = 16
_NW = _NC * _NS


def _ef_body(x_ref, std_ref, mean_ref, out_ref):
    vm = x_ref[:, 0:1] * std_ref[0, 0] + mean_ref[0, 0]
    va = (x_ref[:, 1:2] * std_ref[0, 1] + mean_ref[0, 1]) * (jnp.pi / 180.0)
    out_ref[:, 0:1] = vm * jnp.cos(va)
    out_ref[:, 1:2] = vm * jnp.sin(va)


def _ef_table(x, xystd, xymean):
    n = x.shape[0]
    bn = 10000
    return pl.pallas_call(
        _ef_body,
        grid=(n // bn,),
        in_specs=[
            pl.BlockSpec((bn, 6), lambda i: (i, 0)),
            pl.BlockSpec((1, 6), lambda i: (0, 0)),
            pl.BlockSpec((1, 6), lambda i: (0, 0)),
        ],
        out_specs=pl.BlockSpec((bn, 2), lambda i: (i, 0)),
        out_shape=jax.ShapeDtypeStruct((n, 2), jnp.float32),
    )(x, xystd, xymean)


def _sc_edge_pass(ep_tab, eif, ra_all, xa_all, prm_f, prm_i):
    n = ep_tab.shape[0]
    e_total = eif.shape[0] // 2
    epw = e_total // _NW
    c_chunk = 128
    n_full = (epw - 80) // c_chunk
    tail = epw - n_full * c_chunk
    zsl = ((n + _NS * _L - 1) // (_NS * _L)) * _L
    zlast = n - (_NS - 1) * zsl

    mesh = plsc.VectorSubcoreMesh(core_axis_name="c", subcore_axis_name="s")

    @functools.partial(
        pl.kernel,
        out_type=(
            jax.ShapeDtypeStruct((_NC * n,), jnp.float32),
            jax.ShapeDtypeStruct((_NC * n,), jnp.float32),
            jax.ShapeDtypeStruct((_NC * n,), jnp.float32),
            jax.ShapeDtypeStruct((_NC * n,), jnp.float32),
            jax.ShapeDtypeStruct((_NW * _L,), jnp.int32),
        ),
        mesh=mesh,
        compiler_params=pltpu.CompilerParams(needs_layout_passes=False),
        scratch_types=[
            pltpu.VMEM_SHARED((n,), jnp.float32),
            pltpu.VMEM_SHARED((n,), jnp.float32),
            pltpu.VMEM_SHARED((n,), jnp.float32),
            pltpu.VMEM_SHARED((n,), jnp.float32),
            pltpu.VMEM_SHARED((n,), jnp.float32),
            pltpu.VMEM((zsl,), jnp.float32),
            pltpu.VMEM((zsl,), jnp.float32),
            pltpu.VMEM((2, c_chunk), jnp.int32),
            pltpu.VMEM((2, c_chunk), jnp.int32),
            pltpu.VMEM((2, c_chunk), jnp.float32),
            pltpu.VMEM((2, c_chunk), jnp.float32),
            pltpu.VMEM((80,), jnp.int32),
            pltpu.VMEM((80,), jnp.int32),
            pltpu.VMEM((c_chunk,), jnp.float32),
            pltpu.VMEM((c_chunk,), jnp.float32),
            pltpu.VMEM((c_chunk,), jnp.float32),
            pltpu.VMEM((c_chunk,), jnp.float32),
            pltpu.VMEM((c_chunk,), jnp.float32),
            pltpu.VMEM((c_chunk,), jnp.float32),
            pltpu.VMEM((_L,), jnp.int32),
            pltpu.VMEM((8,), jnp.float32),
            pltpu.VMEM((8,), jnp.int32),
            pltpu.SemaphoreType.DMA,
            pltpu.SemaphoreType.DMA,
            pltpu.SemaphoreType.DMA,
        ],
    )
    def sc_kernel(ep_hbm, eif_hbm, ra_hbm, xa_hbm,
                  prmf_hbm, prmi_hbm,
                  aggpo_hbm, aggqo_hbm, aggpd_hbm, aggqd_hbm, flags_hbm,
                  ep_sh, po_sh, qo_sh, pd_sh, qd_sh, zbuf, bbuf,
                  ii2, jj2, ra2, xa2, ii_t, jj_t, gvi, gvj,
                  p1_v, q1_v, p2_v, q2_v, flags_v, prmf_v, prmi_v,
                  semg, seml0, seml1):
        cc = lax.axis_index("c")
        ss = lax.axis_index("s")
        wid = ss * _NC + cc
        z16i = jnp.zeros((_L,), jnp.int32)
        one16i = z16i + 1
        zero16f = jnp.zeros((_L,), jnp.float32)
        iota16 = lax.iota(jnp.int32, _L)
        seml = (seml0, seml1)

        @pl.loop(0, zsl // _L)
        def _(zi):
            zbuf[pl.ds(zi * _L, _L)] = zero16f

        off = ss * zsl

        @pl.when(ss < _NS - 1)
        def _():
            pltpu.sync_copy(ep_hbm.at[pl.ds(off, zsl)], bbuf)
            pltpu.sync_copy(bbuf, ep_sh.at[pl.ds(off, zsl)])
            pltpu.sync_copy(zbuf, po_sh.at[pl.ds(off, zsl)])
            pltpu.sync_copy(zbuf, qo_sh.at[pl.ds(off, zsl)])
            pltpu.sync_copy(zbuf, pd_sh.at[pl.ds(off, zsl)])
            pltpu.sync_copy(zbuf, qd_sh.at[pl.ds(off, zsl)])

        @pl.when(ss == _NS - 1)
        def _():
            pltpu.sync_copy(ep_hbm.at[pl.ds(off, zlast)], bbuf.at[pl.ds(0, zlast)])
            pltpu.sync_copy(bbuf.at[pl.ds(0, zlast)], ep_sh.at[pl.ds(off, zlast)])
            pltpu.sync_copy(zbuf.at[pl.ds(0, zlast)], po_sh.at[pl.ds(off, zlast)])
            pltpu.sync_copy(zbuf.at[pl.ds(0, zlast)], qo_sh.at[pl.ds(off, zlast)])
            pltpu.sync_copy(zbuf.at[pl.ds(0, zlast)], pd_sh.at[pl.ds(off, zlast)])
            pltpu.sync_copy(zbuf.at[pl.ds(0, zlast)], qd_sh.at[pl.ds(off, zlast)])

        pltpu.sync_copy(prmf_hbm, prmf_v)
        pltpu.sync_copy(prmi_hbm, prmi_v)
        estd0 = plsc.load_gather(prmf_v, [one16i])
        estd1 = plsc.load_gather(prmf_v, [one16i + 1])
        emn0 = plsc.load_gather(prmf_v, [one16i + 2])
        emn1 = plsc.load_gather(prmf_v, [one16i + 3])
        a0 = plsc.load_gather(prmi_v, [one16i])
        b0 = plsc.load_gather(prmi_v, [one16i + 1])
        flags_v[...] = z16i

        plsc.subcore_barrier()

        base_w = wid * epw

        def compute_scatter(iref, jref, raref, xaref, cs):
            g1 = pltpu.async_copy(ep_sh.at[iref], gvi.at[pl.ds(0, cs)], semg)
            g2 = pltpu.async_copy(ep_sh.at[jref], gvj.at[pl.ds(0, cs)], semg)
            g1.wait()
            g2.wait()
            for kk in range(cs // _L):
                sl = pl.ds(kk * _L, _L)
                src16 = iref[sl]
                dst16 = jref[sl]
                e_i, f_i = plsc.unpack(
                    plsc.bitcast(gvi[sl], jnp.bfloat16),
                    format=plsc.PackFormat.INTERLEAVED)
                e_j, f_j = plsc.unpack(
                    plsc.bitcast(gvj[sl], jnp.bfloat16),
                    format=plsc.PackFormat.INTERLEAVED)
                r = raref[sl] * estd0 + emn0
                xx = xaref[sl] * estd1 + emn1
                inv = 1.0 / (r * r + xx * xx)
                g = r * inv
                b = -(xx * inv)
                eef = e_i * e_j + f_i * f_j
                tb = f_i * e_j - e_i * f_j
                ta = eef - e_i * e_i - f_i * f_i
                ta2 = eef - e_j * e_j - f_j * f_j
                gtb = g * tb
                btb = b * tb
                p1_v[sl] = g * ta + btb
                q1_v[sl] = gtb - b * ta
                p2_v[sl] = g * ta2 - btb
                q2_v[sl] = -gtb - b * ta2
                hit = (src16 == b0) & (dst16 == a0)
                flags_v[...] = flags_v[...] | jnp.where(hit, 1, 0)
            s1 = pltpu.async_copy(p1_v.at[pl.ds(0, cs)], po_sh.at[iref],
                                  semg, add=True)
            s2 = pltpu.async_copy(q1_v.at[pl.ds(0, cs)], qo_sh.at[iref],
                                  semg, add=True)
            s3 = pltpu.async_copy(p2_v.at[pl.ds(0, cs)], pd_sh.at[jref],
                                  semg, add=True)
            s4 = pltpu.async_copy(q2_v.at[pl.ds(0, cs)], qd_sh.at[jref],
                                  semg, add=True)
            s1.wait()
            s2.wait()
            s3.wait()
            s4.wait()

        pltpu.sync_copy(eif_hbm.at[pl.ds(base_w, tail)], ii_t)
        pltpu.sync_copy(eif_hbm.at[pl.ds(e_total + base_w, tail)], jj_t)
        pltpu.sync_copy(ra_hbm.at[pl.ds(base_w, tail)],
                        ra2.at[0].at[pl.ds(0, tail)])
        pltpu.sync_copy(xa_hbm.at[pl.ds(base_w, tail)],
                        xa2.at[0].at[pl.ds(0, tail)])
        compute_scatter(ii_t, jj_t, ra2.at[0], xa2.at[0], tail)

        base0 = base_w + tail

        def issue_linear(k, bb):
            base = base0 + k * c_chunk
            pltpu.async_copy(eif_hbm.at[pl.ds(base, c_chunk)], ii2.at[bb],
                             seml[bb])
            pltpu.async_copy(eif_hbm.at[pl.ds(e_total + base, c_chunk)],
                             jj2.at[bb], seml[bb])
            pltpu.async_copy(ra_hbm.at[pl.ds(base, c_chunk)], ra2.at[bb],
                             seml[bb])
            pltpu.async_copy(xa_hbm.at[pl.ds(base, c_chunk)], xa2.at[bb],
                             seml[bb])

        def drain_linear(bb):
            pltpu.make_async_copy(eif_hbm.at[pl.ds(0, c_chunk)], ii2.at[bb],
                                  seml[bb]).wait()
            pltpu.make_async_copy(eif_hbm.at[pl.ds(0, c_chunk)], jj2.at[bb],
                                  seml[bb]).wait()
            pltpu.make_async_copy(ra_hbm.at[pl.ds(0, c_chunk)], ra2.at[bb],
                                  seml[bb]).wait()
            pltpu.make_async_copy(ra_hbm.at[pl.ds(0, c_chunk)], xa2.at[bb],
                                  seml[bb]).wait()

        issue_linear(0, 0)

        @pl.loop(0, n_full // 2)
        def _(it):
            for bb in range(2):
                k = it * 2 + bb
                drain_linear(bb)

                @pl.when(k + 1 < n_full)
                def _():
                    issue_linear(k + 1, 1 - bb)

                compute_scatter(ii2.at[bb], jj2.at[bb], ra2.at[bb],
                                xa2.at[bb], c_chunk)

        pltpu.sync_copy(flags_v, flags_hbm.at[pl.ds(wid * _L, _L)])
        plsc.subcore_barrier()

        hoff = cc * n + off

        @pl.when(ss < _NS - 1)
        def _():
            pltpu.sync_copy(po_sh.at[pl.ds(off, zsl)], bbuf)
            pltpu.sync_copy(bbuf, aggpo_hbm.at[pl.ds(hoff, zsl)])
            pltpu.sync_copy(qo_sh.at[pl.ds(off, zsl)], bbuf)
            pltpu.sync_copy(bbuf, aggqo_hbm.at[pl.ds(hoff, zsl)])
            pltpu.sync_copy(pd_sh.at[pl.ds(off, zsl)], bbuf)
            pltpu.sync_copy(bbuf, aggpd_hbm.at[pl.ds(hoff, zsl)])
            pltpu.sync_copy(qd_sh.at[pl.ds(off, zsl)], bbuf)
            pltpu.sync_copy(bbuf, aggqd_hbm.at[pl.ds(hoff, zsl)])

        @pl.when(ss == _NS - 1)
        def _():
            pltpu.sync_copy(po_sh.at[pl.ds(off, zlast)], bbuf.at[pl.ds(0, zlast)])
            pltpu.sync_copy(bbuf.at[pl.ds(0, zlast)], aggpo_hbm.at[pl.ds(hoff, zlast)])
            pltpu.sync_copy(qo_sh.at[pl.ds(off, zlast)], bbuf.at[pl.ds(0, zlast)])
            pltpu.sync_copy(bbuf.at[pl.ds(0, zlast)], aggqo_hbm.at[pl.ds(hoff, zlast)])
            pltpu.sync_copy(pd_sh.at[pl.ds(off, zlast)], bbuf.at[pl.ds(0, zlast)])
            pltpu.sync_copy(bbuf.at[pl.ds(0, zlast)], aggpd_hbm.at[pl.ds(hoff, zlast)])
            pltpu.sync_copy(qd_sh.at[pl.ds(off, zlast)], bbuf.at[pl.ds(0, zlast)])
            pltpu.sync_copy(bbuf.at[pl.ds(0, zlast)], aggqd_hbm.at[pl.ds(hoff, zlast)])

    return sc_kernel(ep_tab, eif, ra_all, xa_all, prm_f, prm_i)


def _loss_body(w_ref, flags_ref, std_ref, mean_ref, out_ref, *, n_nodes):
    pid = pl.program_id(0)

    @pl.when(pid == 0)
    def _():
        out_ref[...] = jnp.zeros((1, 1), jnp.float32)

    wb = w_ref[...]
    xs = wb[:, 0:6]
    ys = wb[:, 6:12]
    mse_part = jnp.sum((xs - ys) ** 2, keepdims=True)
    wdup = jnp.where(jnp.any(flags_ref[...] != 0), 0.0, 1.0)
    agg_p = wb[:, 12:13] + wb[:, 13:14] + wdup * (wb[:, 16:17] + wb[:, 17:18])
    agg_q = wb[:, 14:15] + wb[:, 15:16] + wdup * (wb[:, 18:19] + wb[:, 19:20])
    p_inj = wb[:, 2:3] * std_ref[0, 2] + mean_ref[0, 2]
    q_inj = wb[:, 3:4] * std_ref[0, 3] + mean_ref[0, 3]
    dp = p_inj - agg_p
    dq = q_inj - agg_q
    imb_part = jnp.sum(dp * dp + dq * dq, keepdims=True)
    out_ref[...] += (0.5 / (6.0 * n_nodes)) * mse_part + \
        (0.5 * 0.02 / n_nodes) * imb_part


def _loss_kernel(w, flags, xystd, xymean):
    n = w.shape[0]
    bn = 10000
    return pl.pallas_call(
        functools.partial(_loss_body, n_nodes=n),
        grid=(n // bn,),
        in_specs=[
            pl.BlockSpec((bn, 20), lambda i: (i, 0)),
            pl.BlockSpec((_NW * _L // 128, 128), lambda i: (0, 0)),
            pl.BlockSpec((1, 6), lambda i: (0, 0)),
            pl.BlockSpec((1, 6), lambda i: (0, 0)),
        ],
        out_specs=pl.BlockSpec((1, 1), lambda i: (0, 0)),
        out_shape=jax.ShapeDtypeStruct((1, 1), jnp.float32),
    )(w, flags, xystd, xymean)


def kernel(x, edge_index, edge_attr, y, xymean, xystd, edgemean, edgestd):
    ef = _ef_table(x, xystd, xymean)
    zf1 = jnp.zeros((1,), jnp.float32)
    prm_f = jnp.concatenate([zf1, edgestd.reshape(2), edgemean.reshape(2),
                             jnp.zeros((3,), jnp.float32)])
    zi1 = jnp.zeros((1,), jnp.int32)
    prm_i = jnp.concatenate([zi1, edge_index[0, 0:1], edge_index[1, 0:1],
                             jnp.zeros((5,), jnp.int32)])
    eb = lax.bitcast_convert_type(
        ef[:, 0].astype(jnp.bfloat16), jnp.uint16).astype(jnp.uint32)
    fb = lax.bitcast_convert_type(
        ef[:, 1].astype(jnp.bfloat16), jnp.uint16).astype(jnp.uint32)
    ep = lax.bitcast_convert_type(eb | (fb << jnp.uint32(16)), jnp.float32)
    aggpo, aggqo, aggpd, aggqd, flags = _sc_edge_pass(
        ep, edge_index.reshape(-1), edge_attr[:, 0], edge_attr[:, 1],
        prm_f, prm_i)
    w = jnp.concatenate(
        [x, y,
         aggpo.reshape(_NC, -1).T, aggqo.reshape(_NC, -1).T,
         aggpd.reshape(_NC, -1).T, aggqd.reshape(_NC, -1).T], axis=1)
    loss = _loss_kernel(w, flags.reshape(_NW * _L // 128, 128),
                        xystd, xymean)
    return loss[0, 0]


# speedup vs baseline: 3.8040x; 2.2456x over previous
import functools

import jax
import jax.numpy as jnp
from jax import lax
from jax.experimental import pallas as pl
from jax.experimental.pallas import tpu as pltpu
from jax.experimental.pallas import tpu_sc as plsc

_NC = 2
_NS = 16
_L = 16
_NW = _NC * _NS


def _ef_mse_body(xt_ref, yt_ref, std_ref, mean_ref, ef_ref, mse_ref):
    vm = xt_ref[0:1, :] * std_ref[0, 0] + mean_ref[0, 0]
    va = (xt_ref[1:2, :] * std_ref[0, 1] + mean_ref[0, 1]) * (jnp.pi / 180.0)
    ef_ref[0:1, :] = vm * jnp.cos(va)
    ef_ref[1:2, :] = vm * jnp.sin(va)
    d = xt_ref[...] - yt_ref[...]
    mse_ref[...] = jnp.sum(d * d, keepdims=True)


def _ef_mse_kernel(xt, yt, xystd, xymean):
    n = xt.shape[1]
    return pl.pallas_call(
        _ef_mse_body,
        out_shape=(jax.ShapeDtypeStruct((2, n), jnp.float32),
                   jax.ShapeDtypeStruct((1, 1), jnp.float32)),
    )(xt, yt, xystd, xymean)


def _sc_edge_pass(ep_tab, eif, ra_all, xa_all, prm_f, prm_i):
    n = ep_tab.shape[0]
    e_total = eif.shape[0] // 2
    epw = e_total // _NW
    c_chunk = 128
    n_full = (epw - 80) // c_chunk
    tail = epw - n_full * c_chunk
    zsl = ((n + _NS * _L - 1) // (_NS * _L)) * _L
    zlast = n - (_NS - 1) * zsl

    mesh = plsc.VectorSubcoreMesh(core_axis_name="c", subcore_axis_name="s")

    @functools.partial(
        pl.kernel,
        out_type=(
            jax.ShapeDtypeStruct((_NC * n,), jnp.float32),
            jax.ShapeDtypeStruct((_NC * n,), jnp.float32),
            jax.ShapeDtypeStruct((_NC * n,), jnp.float32),
            jax.ShapeDtypeStruct((_NC * n,), jnp.float32),
            jax.ShapeDtypeStruct((_NW * _L,), jnp.int32),
        ),
        mesh=mesh,
        compiler_params=pltpu.CompilerParams(needs_layout_passes=False),
        scratch_types=[
            pltpu.VMEM_SHARED((n,), jnp.float32),
            pltpu.VMEM_SHARED((n,), jnp.float32),
            pltpu.VMEM_SHARED((n,), jnp.float32),
            pltpu.VMEM_SHARED((n,), jnp.float32),
            pltpu.VMEM_SHARED((n,), jnp.float32),
            pltpu.VMEM((zsl,), jnp.float32),
            pltpu.VMEM((zsl,), jnp.float32),
            pltpu.VMEM((2, c_chunk), jnp.int32),
            pltpu.VMEM((2, c_chunk), jnp.int32),
            pltpu.VMEM((2, c_chunk), jnp.float32),
            pltpu.VMEM((2, c_chunk), jnp.float32),
            pltpu.VMEM((80,), jnp.int32),
            pltpu.VMEM((80,), jnp.int32),
            pltpu.VMEM((c_chunk,), jnp.float32),
            pltpu.VMEM((c_chunk,), jnp.float32),
            pltpu.VMEM((c_chunk,), jnp.float32),
            pltpu.VMEM((c_chunk,), jnp.float32),
            pltpu.VMEM((c_chunk,), jnp.float32),
            pltpu.VMEM((c_chunk,), jnp.float32),
            pltpu.VMEM((_L,), jnp.int32),
            pltpu.VMEM((8,), jnp.float32),
            pltpu.VMEM((8,), jnp.int32),
            pltpu.SemaphoreType.DMA,
            pltpu.SemaphoreType.DMA,
            pltpu.SemaphoreType.DMA,
        ],
    )
    def sc_kernel(ep_hbm, eif_hbm, ra_hbm, xa_hbm,
                  prmf_hbm, prmi_hbm,
                  aggpo_hbm, aggqo_hbm, aggpd_hbm, aggqd_hbm, flags_hbm,
                  ep_sh, po_sh, qo_sh, pd_sh, qd_sh, zbuf, bbuf,
                  ii2, jj2, ra2, xa2, ii_t, jj_t, gvi, gvj,
                  p1_v, q1_v, p2_v, q2_v, flags_v, prmf_v, prmi_v,
                  semg, seml0, seml1):
        cc = lax.axis_index("c")
        ss = lax.axis_index("s")
        wid = ss * _NC + cc
        z16i = jnp.zeros((_L,), jnp.int32)
        one16i = z16i + 1
        zero16f = jnp.zeros((_L,), jnp.float32)
        iota16 = lax.iota(jnp.int32, _L)
        seml = (seml0, seml1)

        @pl.loop(0, zsl // _L)
        def _(zi):
            zbuf[pl.ds(zi * _L, _L)] = zero16f

        off = ss * zsl

        @pl.when(ss < _NS - 1)
        def _():
            pltpu.sync_copy(ep_hbm.at[pl.ds(off, zsl)], bbuf)
            pltpu.sync_copy(bbuf, ep_sh.at[pl.ds(off, zsl)])
            pltpu.sync_copy(zbuf, po_sh.at[pl.ds(off, zsl)])
            pltpu.sync_copy(zbuf, qo_sh.at[pl.ds(off, zsl)])
            pltpu.sync_copy(zbuf, pd_sh.at[pl.ds(off, zsl)])
            pltpu.sync_copy(zbuf, qd_sh.at[pl.ds(off, zsl)])

        @pl.when(ss == _NS - 1)
        def _():
            pltpu.sync_copy(ep_hbm.at[pl.ds(off, zlast)], bbuf.at[pl.ds(0, zlast)])
            pltpu.sync_copy(bbuf.at[pl.ds(0, zlast)], ep_sh.at[pl.ds(off, zlast)])
            pltpu.sync_copy(zbuf.at[pl.ds(0, zlast)], po_sh.at[pl.ds(off, zlast)])
            pltpu.sync_copy(zbuf.at[pl.ds(0, zlast)], qo_sh.at[pl.ds(off, zlast)])
            pltpu.sync_copy(zbuf.at[pl.ds(0, zlast)], pd_sh.at[pl.ds(off, zlast)])
            pltpu.sync_copy(zbuf.at[pl.ds(0, zlast)], qd_sh.at[pl.ds(off, zlast)])

        pltpu.sync_copy(prmf_hbm, prmf_v)
        pltpu.sync_copy(prmi_hbm, prmi_v)
        estd0 = plsc.load_gather(prmf_v, [one16i])
        estd1 = plsc.load_gather(prmf_v, [one16i + 1])
        emn0 = plsc.load_gather(prmf_v, [one16i + 2])
        emn1 = plsc.load_gather(prmf_v, [one16i + 3])
        a0 = plsc.load_gather(prmi_v, [one16i])
        b0 = plsc.load_gather(prmi_v, [one16i + 1])
        flags_v[...] = z16i

        plsc.subcore_barrier()

        base_w = wid * epw

        def compute_scatter(iref, jref, raref, xaref, cs):
            g1 = pltpu.async_copy(ep_sh.at[iref], gvi.at[pl.ds(0, cs)], semg)
            g2 = pltpu.async_copy(ep_sh.at[jref], gvj.at[pl.ds(0, cs)], semg)
            g1.wait()
            g2.wait()
            for kk in range(cs // _L):
                sl = pl.ds(kk * _L, _L)
                src16 = iref[sl]
                dst16 = jref[sl]
                e_i, f_i = plsc.unpack(
                    plsc.bitcast(gvi[sl], jnp.bfloat16),
                    format=plsc.PackFormat.INTERLEAVED)
                e_j, f_j = plsc.unpack(
                    plsc.bitcast(gvj[sl], jnp.bfloat16),
                    format=plsc.PackFormat.INTERLEAVED)
                r = raref[sl] * estd0 + emn0
                xx = xaref[sl] * estd1 + emn1
                inv = 1.0 / (r * r + xx * xx)
                g = r * inv
                b = -(xx * inv)
                eef = e_i * e_j + f_i * f_j
                tb = f_i * e_j - e_i * f_j
                ta = eef - e_i * e_i - f_i * f_i
                ta2 = eef - e_j * e_j - f_j * f_j
                gtb = g * tb
                btb = b * tb
                p1_v[sl] = g * ta + btb
                q1_v[sl] = gtb - b * ta
                p2_v[sl] = g * ta2 - btb
                q2_v[sl] = -gtb - b * ta2
                hit = (src16 == b0) & (dst16 == a0)
                flags_v[...] = flags_v[...] | jnp.where(hit, 1, 0)
            s1 = pltpu.async_copy(p1_v.at[pl.ds(0, cs)], po_sh.at[iref],
                                  semg, add=True)
            s2 = pltpu.async_copy(q1_v.at[pl.ds(0, cs)], qo_sh.at[iref],
                                  semg, add=True)
            s3 = pltpu.async_copy(p2_v.at[pl.ds(0, cs)], pd_sh.at[jref],
                                  semg, add=True)
            s4 = pltpu.async_copy(q2_v.at[pl.ds(0, cs)], qd_sh.at[jref],
                                  semg, add=True)
            s1.wait()
            s2.wait()
            s3.wait()
            s4.wait()

        pltpu.sync_copy(eif_hbm.at[pl.ds(base_w, tail)], ii_t)
        pltpu.sync_copy(eif_hbm.at[pl.ds(e_total + base_w, tail)], jj_t)
        pltpu.sync_copy(ra_hbm.at[pl.ds(base_w, tail)],
                        ra2.at[0].at[pl.ds(0, tail)])
        pltpu.sync_copy(xa_hbm.at[pl.ds(base_w, tail)],
                        xa2.at[0].at[pl.ds(0, tail)])
        compute_scatter(ii_t, jj_t, ra2.at[0], xa2.at[0], tail)

        base0 = base_w + tail

        def issue_linear(k, bb):
            base = base0 + k * c_chunk
            pltpu.async_copy(eif_hbm.at[pl.ds(base, c_chunk)], ii2.at[bb],
                             seml[bb])
            pltpu.async_copy(eif_hbm.at[pl.ds(e_total + base, c_chunk)],
                             jj2.at[bb], seml[bb])
            pltpu.async_copy(ra_hbm.at[pl.ds(base, c_chunk)], ra2.at[bb],
                             seml[bb])
            pltpu.async_copy(xa_hbm.at[pl.ds(base, c_chunk)], xa2.at[bb],
                             seml[bb])

        def drain_linear(bb):
            pltpu.make_async_copy(eif_hbm.at[pl.ds(0, c_chunk)], ii2.at[bb],
                                  seml[bb]).wait()
            pltpu.make_async_copy(eif_hbm.at[pl.ds(0, c_chunk)], jj2.at[bb],
                                  seml[bb]).wait()
            pltpu.make_async_copy(ra_hbm.at[pl.ds(0, c_chunk)], ra2.at[bb],
                                  seml[bb]).wait()
            pltpu.make_async_copy(ra_hbm.at[pl.ds(0, c_chunk)], xa2.at[bb],
                                  seml[bb]).wait()

        issue_linear(0, 0)

        @pl.loop(0, n_full // 2)
        def _(it):
            for bb in range(2):
                k = it * 2 + bb
                drain_linear(bb)

                @pl.when(k + 1 < n_full)
                def _():
                    issue_linear(k + 1, 1 - bb)

                compute_scatter(ii2.at[bb], jj2.at[bb], ra2.at[bb],
                                xa2.at[bb], c_chunk)

        pltpu.sync_copy(flags_v, flags_hbm.at[pl.ds(wid * _L, _L)])
        plsc.subcore_barrier()

        hoff = cc * n + off

        @pl.when(ss < _NS - 1)
        def _():
            pltpu.sync_copy(po_sh.at[pl.ds(off, zsl)], bbuf)
            pltpu.sync_copy(bbuf, aggpo_hbm.at[pl.ds(hoff, zsl)])
            pltpu.sync_copy(qo_sh.at[pl.ds(off, zsl)], bbuf)
            pltpu.sync_copy(bbuf, aggqo_hbm.at[pl.ds(hoff, zsl)])
            pltpu.sync_copy(pd_sh.at[pl.ds(off, zsl)], bbuf)
            pltpu.sync_copy(bbuf, aggpd_hbm.at[pl.ds(hoff, zsl)])
            pltpu.sync_copy(qd_sh.at[pl.ds(off, zsl)], bbuf)
            pltpu.sync_copy(bbuf, aggqd_hbm.at[pl.ds(hoff, zsl)])

        @pl.when(ss == _NS - 1)
        def _():
            pltpu.sync_copy(po_sh.at[pl.ds(off, zlast)], bbuf.at[pl.ds(0, zlast)])
            pltpu.sync_copy(bbuf.at[pl.ds(0, zlast)], aggpo_hbm.at[pl.ds(hoff, zlast)])
            pltpu.sync_copy(qo_sh.at[pl.ds(off, zlast)], bbuf.at[pl.ds(0, zlast)])
            pltpu.sync_copy(bbuf.at[pl.ds(0, zlast)], aggqo_hbm.at[pl.ds(hoff, zlast)])
            pltpu.sync_copy(pd_sh.at[pl.ds(off, zlast)], bbuf.at[pl.ds(0, zlast)])
            pltpu.sync_copy(bbuf.at[pl.ds(0, zlast)], aggpd_hbm.at[pl.ds(hoff, zlast)])
            pltpu.sync_copy(qd_sh.at[pl.ds(off, zlast)], bbuf.at[pl.ds(0, zlast)])
            pltpu.sync_copy(bbuf.at[pl.ds(0, zlast)], aggqd_hbm.at[pl.ds(hoff, zlast)])

    return sc_kernel(ep_tab, eif, ra_all, xa_all, prm_f, prm_i)


def _loss_body(xt_ref, mse_ref, apo, aqo, apd, aqd, flags_ref,
               std_ref, mean_ref, out_ref, *, n_nodes):
    wdup = jnp.where(jnp.any(flags_ref[...] != 0), 0.0, 1.0)
    agg_p = apo[0:1, :] + apo[1:2, :] + wdup * (apd[0:1, :] + apd[1:2, :])
    agg_q = aqo[0:1, :] + aqo[1:2, :] + wdup * (aqd[0:1, :] + aqd[1:2, :])
    p_inj = xt_ref[2:3, :] * std_ref[0, 2] + mean_ref[0, 2]
    q_inj = xt_ref[3:4, :] * std_ref[0, 3] + mean_ref[0, 3]
    dp = p_inj - agg_p
    dq = q_inj - agg_q
    imb_part = jnp.sum(dp * dp + dq * dq, keepdims=True)
    out_ref[...] = (0.5 / (6.0 * n_nodes)) * mse_ref[...] + \
        (0.5 * 0.02 / n_nodes) * imb_part


def _loss_kernel(xt, mse11, apo, aqo, apd, aqd, flags, xystd, xymean):
    n = xt.shape[1]
    return pl.pallas_call(
        functools.partial(_loss_body, n_nodes=n),
        out_shape=jax.ShapeDtypeStruct((1, 1), jnp.float32),
    )(xt, mse11, apo, aqo, apd, aqd, flags, xystd, xymean)


def kernel(x, edge_index, edge_attr, y, xymean, xystd, edgemean, edgestd):
    xt = x.T
    yt = y.T
    ef2, mse11 = _ef_mse_kernel(xt, yt, xystd, xymean)
    zf1 = jnp.zeros((1,), jnp.float32)
    prm_f = jnp.concatenate([zf1, edgestd.reshape(2), edgemean.reshape(2),
                             jnp.zeros((3,), jnp.float32)])
    zi1 = jnp.zeros((1,), jnp.int32)
    prm_i = jnp.concatenate([zi1, edge_index[0, 0:1], edge_index[1, 0:1],
                             jnp.zeros((5,), jnp.int32)])
    eb = lax.bitcast_convert_type(
        ef2[0].astype(jnp.bfloat16), jnp.uint16).astype(jnp.uint32)
    fb = lax.bitcast_convert_type(
        ef2[1].astype(jnp.bfloat16), jnp.uint16).astype(jnp.uint32)
    ep = lax.bitcast_convert_type(eb | (fb << jnp.uint32(16)), jnp.float32)
    aggpo, aggqo, aggpd, aggqd, flags = _sc_edge_pass(
        ep, edge_index.reshape(-1), edge_attr[:, 0], edge_attr[:, 1],
        prm_f, prm_i)
    loss = _loss_kernel(xt, mse11,
                        aggpo.reshape(_NC, -1), aggqo.reshape(_NC, -1),
                        aggpd.reshape(_NC, -1), aggqd.reshape(_NC, -1),
                        flags.reshape(_NW * _L // 128, 128),
                        xystd, xymean)
    return loss[0, 0]
